# Initial kernel scaffold; baseline (speedup 1.0000x reference)
import jax
import jax.numpy as jnp
from jax.experimental import pallas as pl


def kernel(mu, dictionary, ema_counts):
    raise NotImplementedError("write your pallas kernel here")


# speedup vs baseline: 1.2239x; 1.2239x over previous
import jax
import jax.numpy as jnp
from jax.experimental import pallas as pl

BATCH = 256
DIM_CODES = 8
DICT_SIZE = 1024
DIM_EMBED = 64


def _vq_kernel(mu3_ref, dict_ref, cw_ref, oh_ref):
    for c in range(DIM_CODES):
        x = mu3_ref[:, c, :]
        d = dict_ref[c]
        a2 = jnp.sum(x * x, axis=1, keepdims=True)
        b2 = jnp.sum(d * d, axis=1)[None, :]
        ab = jax.lax.dot_general(
            x, d, (((1,), (1,)), ((), ())),
            preferred_element_type=jnp.float32)
        dist = a2 - 2.0 * ab + b2
        idx = jnp.argmin(dist, axis=1)
        iota = jax.lax.broadcasted_iota(jnp.int32, (BATCH, DICT_SIZE), 1)
        one_hot = (iota == idx[:, None].astype(jnp.int32)).astype(jnp.float32)
        oh_ref[:, c, :] = one_hot
        cw_ref[:, c, :] = jax.lax.dot_general(
            one_hot, d, (((1,), (0,)), ((), ())),
            precision=jax.lax.Precision.HIGHEST,
            preferred_element_type=jnp.float32)


def kernel(mu, dictionary, ema_counts):
    del ema_counts
    batch, cw_dim = mu.shape
    mu3 = mu.reshape(batch, DIM_CODES, DIM_EMBED)
    cw3, one_hot = pl.pallas_call(
        _vq_kernel,
        out_shape=(
            jax.ShapeDtypeStruct((batch, DIM_CODES, DIM_EMBED), jnp.float32),
            jax.ShapeDtypeStruct((batch, DIM_CODES, DICT_SIZE), jnp.float32),
        ),
    )(mu3, dictionary)
    return cw3.reshape(batch, cw_dim), one_hot


# speedup vs baseline: 1.3700x; 1.1194x over previous
import jax
import jax.numpy as jnp
from jax.experimental import pallas as pl

BATCH = 256
DIM_CODES = 8
DICT_SIZE = 1024
DIM_EMBED = 64


def _vq_kernel(mu3_ref, dict_ref, cw_ref, oh_ref):
    for c in range(DIM_CODES):
        x = mu3_ref[:, c, :]
        d = dict_ref[c]
        a2 = jnp.sum(x * x, axis=1, keepdims=True)
        b2 = jnp.sum(d * d, axis=1)[None, :]
        ab = jax.lax.dot_general(
            x, d, (((1,), (1,)), ((), ())),
            preferred_element_type=jnp.float32)
        dist = a2 - 2.0 * ab + b2
        idx = jnp.argmin(dist, axis=1)
        iota = jax.lax.broadcasted_iota(jnp.int32, (BATCH, DICT_SIZE), 1)
        one_hot = (iota == idx[:, None].astype(jnp.int32)).astype(jnp.float32)
        oh_ref[:, c, :] = one_hot
        d1 = d.astype(jnp.bfloat16).astype(jnp.float32)
        r1 = d - d1
        d2 = r1.astype(jnp.bfloat16).astype(jnp.float32)
        d3 = r1 - d2
        dims = (((1,), (0,)), ((), ()))
        cw = jax.lax.dot_general(one_hot, d1, dims,
                                 preferred_element_type=jnp.float32)
        cw += jax.lax.dot_general(one_hot, d2, dims,
                                  preferred_element_type=jnp.float32)
        cw += jax.lax.dot_general(one_hot, d3, dims,
                                  preferred_element_type=jnp.float32)
        cw_ref[:, c, :] = cw


def kernel(mu, dictionary, ema_counts):
    del ema_counts
    batch, cw_dim = mu.shape
    mu3 = mu.reshape(batch, DIM_CODES, DIM_EMBED)
    cw3, one_hot = pl.pallas_call(
        _vq_kernel,
        out_shape=(
            jax.ShapeDtypeStruct((batch, DIM_CODES, DIM_EMBED), jnp.float32),
            jax.ShapeDtypeStruct((batch, DIM_CODES, DICT_SIZE), jnp.float32),
        ),
    )(mu3, dictionary)
    return cw3.reshape(batch, cw_dim), one_hot


# speedup vs baseline: 1.4990x; 1.0942x over previous
import jax
import jax.numpy as jnp
from jax.experimental import pallas as pl
from jax.experimental.pallas import tpu as pltpu

BATCH = 256
DIM_CODES = 8
DICT_SIZE = 1024
DIM_EMBED = 64


def _vq_kernel(mu3_ref, dict_ref, cw_ref, oh_hbm_ref, oh_buf, sems):
    pending = [None, None]
    for c in range(DIM_CODES):
        x = mu3_ref[:, c, :]
        d = dict_ref[c]
        a2 = jnp.sum(x * x, axis=1, keepdims=True)
        b2 = jnp.sum(d * d, axis=1)[None, :]
        ab = jax.lax.dot_general(
            x, d, (((1,), (1,)), ((), ())),
            preferred_element_type=jnp.float32)
        dist = a2 - 2.0 * ab + b2
        idx = jnp.argmin(dist, axis=1)
        iota = jax.lax.broadcasted_iota(jnp.int32, (BATCH, DICT_SIZE), 1)
        one_hot = (iota == idx[:, None].astype(jnp.int32)).astype(jnp.float32)
        buf = c % 2
        if pending[buf] is not None:
            pending[buf].wait()
        oh_buf[buf] = one_hot
        cp = pltpu.make_async_copy(
            oh_buf.at[buf], oh_hbm_ref.at[:, c, :], sems.at[buf])
        cp.start()
        pending[buf] = cp
        d1 = d.astype(jnp.bfloat16).astype(jnp.float32)
        r1 = d - d1
        d2 = r1.astype(jnp.bfloat16).astype(jnp.float32)
        d3 = r1 - d2
        dims = (((1,), (0,)), ((), ()))
        cw = jax.lax.dot_general(one_hot, d1, dims,
                                 preferred_element_type=jnp.float32)
        cw += jax.lax.dot_general(one_hot, d2, dims,
                                  preferred_element_type=jnp.float32)
        cw += jax.lax.dot_general(one_hot, d3, dims,
                                  preferred_element_type=jnp.float32)
        cw_ref[:, c, :] = cw
    for cp in pending:
        cp.wait()


def kernel(mu, dictionary, ema_counts):
    del ema_counts
    batch, cw_dim = mu.shape
    mu3 = mu.reshape(batch, DIM_CODES, DIM_EMBED)
    cw3, one_hot = pl.pallas_call(
        _vq_kernel,
        in_specs=[
            pl.BlockSpec(memory_space=pltpu.MemorySpace.VMEM),
            pl.BlockSpec(memory_space=pltpu.MemorySpace.VMEM),
        ],
        out_specs=(
            pl.BlockSpec(memory_space=pltpu.MemorySpace.VMEM),
            pl.BlockSpec(memory_space=pltpu.MemorySpace.HBM),
        ),
        out_shape=(
            jax.ShapeDtypeStruct((batch, DIM_CODES, DIM_EMBED), jnp.float32),
            jax.ShapeDtypeStruct((batch, DIM_CODES, DICT_SIZE), jnp.float32),
        ),
        scratch_shapes=[
            pltpu.MemorySpace.VMEM((2, BATCH, DICT_SIZE), jnp.float32),
            pltpu.SemaphoreType.DMA((2,)),
        ],
    )(mu3, dictionary)
    return cw3.reshape(batch, cw_dim), one_hot


# speedup vs baseline: 2.0860x; 1.3915x over previous
import jax
import jax.numpy as jnp
from jax.experimental import pallas as pl
from jax.experimental.pallas import tpu as pltpu

BATCH = 256
DIM_CODES = 8
DICT_SIZE = 1024
DIM_EMBED = 64


def _vq_kernel(mu_ref, dict_ref, cw_ref, oh_hbm_ref, oh_buf, sems):
    pending = [None, None]
    for c in range(DIM_CODES):
        x = mu_ref[:, c * DIM_EMBED:(c + 1) * DIM_EMBED]
        d = dict_ref[c]
        a2 = jnp.sum(x * x, axis=1, keepdims=True)
        b2 = jnp.sum(d * d, axis=1)[None, :]
        ab = jax.lax.dot_general(
            x, d, (((1,), (1,)), ((), ())),
            preferred_element_type=jnp.float32)
        dist = a2 - 2.0 * ab + b2
        idx = jnp.argmin(dist, axis=1)
        iota = jax.lax.broadcasted_iota(jnp.int32, (BATCH, DICT_SIZE), 1)
        one_hot = (iota == idx[:, None].astype(jnp.int32)).astype(jnp.float32)
        buf = c % 2
        if pending[buf] is not None:
            pending[buf].wait()
        oh_buf[buf] = one_hot
        cp = pltpu.make_async_copy(
            oh_buf.at[buf], oh_hbm_ref.at[:, c, :], sems.at[buf])
        cp.start()
        pending[buf] = cp
        d1 = d.astype(jnp.bfloat16).astype(jnp.float32)
        r1 = d - d1
        d2 = r1.astype(jnp.bfloat16).astype(jnp.float32)
        d3 = r1 - d2
        dims = (((1,), (0,)), ((), ()))
        cw = jax.lax.dot_general(one_hot, d1, dims,
                                 preferred_element_type=jnp.float32)
        cw += jax.lax.dot_general(one_hot, d2, dims,
                                  preferred_element_type=jnp.float32)
        cw += jax.lax.dot_general(one_hot, d3, dims,
                                  preferred_element_type=jnp.float32)
        cw_ref[:, c * DIM_EMBED:(c + 1) * DIM_EMBED] = cw
    for cp in pending:
        cp.wait()


def kernel(mu, dictionary, ema_counts):
    del ema_counts
    batch, cw_dim = mu.shape
    cw, one_hot = pl.pallas_call(
        _vq_kernel,
        in_specs=[
            pl.BlockSpec(memory_space=pltpu.MemorySpace.VMEM),
            pl.BlockSpec(memory_space=pltpu.MemorySpace.VMEM),
        ],
        out_specs=(
            pl.BlockSpec(memory_space=pltpu.MemorySpace.VMEM),
            pl.BlockSpec(memory_space=pltpu.MemorySpace.HBM),
        ),
        out_shape=(
            jax.ShapeDtypeStruct((batch, cw_dim), jnp.float32),
            jax.ShapeDtypeStruct((batch, DIM_CODES, DICT_SIZE), jnp.float32),
        ),
        scratch_shapes=[
            pltpu.MemorySpace.VMEM((2, BATCH, DICT_SIZE), jnp.float32),
            pltpu.SemaphoreType.DMA((2,)),
        ],
    )(mu, dictionary)
    return cw, one_hot
